# baseline (speedup 1.0000x reference)
import functools

import jax
import jax.numpy as jnp
from jax import lax
from jax.experimental import pallas as pl
from jax.experimental.pallas import tpu as pltpu
from jax.experimental.pallas import tpu_sc as plsc

N_ENTITY = 10000
N_REL = 12
DIM = 128
DW = 144
NUM_BASES = 8
N_EDGES = 320000
B = 64
S = 20

NW = 32
CHUNK = 128
RPT = 80
E_PAD = NW * RPT * CHUNK
EROWS = E_PAD // CHUNK
ER_IN = N_EDGES // CHUNK
SRows = 16
SUB = 64
NBUF = 2
NP = 10112
ROWS_PER_TILE = NP // 16


def _w_body(comp_ref, basis_ref, w_ref):
    bas = basis_ref[...]
    for r in range(N_REL):
        acc = comp_ref[r, 0] * bas[0]
        for b in range(1, NUM_BASES):
            acc = acc + comp_ref[r, b] * bas[b]
        w_ref[r, :, :DIM] = acc
        w_ref[r, :, DIM:] = jnp.ones((acc.shape[0], DW - DIM), jnp.float32)


def _build_w(comp, basis):
    BN = 1000
    return pl.pallas_call(
        _w_body,
        grid=(N_ENTITY // BN,),
        in_specs=[
            pl.BlockSpec(memory_space=pltpu.SMEM),
            pl.BlockSpec((NUM_BASES, BN, DIM), lambda i: (0, i, 0)),
        ],
        out_specs=pl.BlockSpec((N_REL, BN, DW), lambda i: (0, i, 0)),
        out_shape=jax.ShapeDtypeStruct((N_REL, N_ENTITY, DW), jnp.float32),
    )(comp, basis)


def _prep_body(ei_ref, et_ref, fidx_ref, dstp_ref):
    fidx_ref[0:ER_IN] = et_ref[...] * N_ENTITY + ei_ref[0]
    fidx_ref[ER_IN:EROWS] = jnp.zeros((EROWS - ER_IN, CHUNK), jnp.int32)
    dstp_ref[0:ER_IN] = ei_ref[1]
    dstp_ref[ER_IN:EROWS] = jnp.full((EROWS - ER_IN, CHUNK), N_ENTITY,
                                     jnp.int32)


def _prep_edges(ei, et):
    return pl.pallas_call(
        _prep_body,
        out_shape=[
            jax.ShapeDtypeStruct((EROWS, CHUNK), jnp.int32),
            jax.ShapeDtypeStruct((EROWS, CHUNK), jnp.int32),
        ],
    )(ei, et)


def _sc_agg_body(fidx_h, dst_h, w_h, za_h,
                 agg_out,
                 fidx_v, dst_v, fr0, fr1, dr0, dr1, rw0, rw1,
                 agg_s, sg0, sg1, ss0, ss1):
    cid = lax.axis_index("c")
    sid = lax.axis_index("s")
    wid = cid * 16 + sid

    FR = [fr0, fr1]
    DR = [dr0, dr1]
    RW = [rw0, rw1]
    SG = [sg0, sg1]
    SS = [ss0, ss1]

    off = sid * ROWS_PER_TILE
    pltpu.sync_copy(za_h.at[pl.ds(off, ROWS_PER_TILE)],
                    agg_s.at[pl.ds(off, ROWS_PER_TILE)])

    rbase = wid * RPT
    NSUB = SRows * CHUNK // SUB

    plsc.subcore_barrier()

    def copy_idx(s2, k, cb):
        r = s2 // 2
        for j in range(SUB // 16):
            FR[k][pl.ds(16 * j, 16)] = fidx_v[r, pl.ds(cb + 16 * j, 16)]
            DR[k][pl.ds(16 * j, 16)] = dst_v[r, pl.ds(cb + 16 * j, 16)]

    def gstart(k):
        pltpu.async_copy(w_h.at[FR[k]], RW[k], SG[k])

    def gwait(k):
        pltpu.make_async_copy(w_h.at[FR[k]], RW[k], SG[k]).wait()

    def sstart(k):
        pltpu.async_copy(RW[k], agg_s.at[DR[k]], SS[k], add=True)

    def swait(k):
        pltpu.make_async_copy(RW[k], agg_s.at[DR[k]], SS[k]).wait()

    for ph in range(RPT // SRows):
        pltpu.sync_copy(fidx_h.at[pl.ds(rbase + ph * SRows, SRows)], fidx_v)
        pltpu.sync_copy(dst_h.at[pl.ds(rbase + ph * SRows, SRows)], dst_v)

        copy_idx(jnp.int32(0), 0, 0)
        gstart(0)

        def pair(q, carry):
            for k in range(NBUF):
                s = NBUF * q + k
                gwait(k)
                sstart(k)
                kp = (k + 1) % NBUF

                @pl.when(s + 1 < NSUB)
                def _():
                    @pl.when(s >= 1)
                    def _():
                        swait(kp)
                    copy_idx(s + 1, kp, kp * SUB)
                    gstart(kp)
            return carry

        lax.fori_loop(0, NSUB // NBUF, pair, 0)
        for k in range(NBUF):
            swait(k)

    plsc.subcore_barrier()

    pltpu.sync_copy(agg_s.at[pl.ds(off, ROWS_PER_TILE)],
                    agg_out.at[cid, pl.ds(off, ROWS_PER_TILE)])


_sc_agg = functools.partial(
    pl.kernel,
    out_type=jax.ShapeDtypeStruct((2, NP, DW), jnp.float32),
    mesh=plsc.VectorSubcoreMesh(core_axis_name="c", subcore_axis_name="s"),
    compiler_params=pltpu.CompilerParams(use_tc_tiling_on_sc=False),
    scratch_types=(
        [
            pltpu.VMEM((SRows, CHUNK), jnp.int32),
            pltpu.VMEM((SRows, CHUNK), jnp.int32),
        ]
        + [pltpu.VMEM((SUB,), jnp.int32) for _ in range(2 * NBUF)]
        + [pltpu.VMEM((SUB, DW), jnp.float32) for _ in range(NBUF)]
        + [pltpu.VMEM_SHARED((NP, DW), jnp.float32)]
        + [pltpu.SemaphoreType.DMA for _ in range(2 * NBUF)]
    ),
)(_sc_agg_body)


def _tail_body(seed_ref, lab_ref, agg_ref, root_ref, rbias_ref, a_ref, b_ref,
               bias_ref, scores_ref, loss_ref, nodes_ref, h_ref, e_ref):
    v = agg_ref[0, 0:N_ENTITY, 0:DIM] + agg_ref[1, 0:N_ENTITY, 0:DIM]
    deg = agg_ref[0, 0:N_ENTITY, DIM:DIM + 1] + agg_ref[1, 0:N_ENTITY,
                                                        DIM:DIM + 1]
    nodes = v / jnp.maximum(deg, 1.0) + root_ref[...] + rbias_ref[...]
    nodes_ref[...] = nodes

    def gather_b(b, carry):
        for s in range(S):
            idx = seed_ref[b, s]
            h_ref[s, pl.ds(b, 1), :] = nodes_ref[pl.ds(idx, 1), :]
        return carry
    lax.fori_loop(0, B, gather_b, 0)

    e_ref[...] = jnp.full((B, 32), -1e30, jnp.float32)
    a_mat = a_ref[...]
    b_row = b_ref[...]
    for s in range(S):
        hs = h_ref[s]
        t = jnp.tanh(lax.dot_general(
            hs, a_mat, (((1,), (0,)), ((), ())),
            preferred_element_type=jnp.float32,
            precision=lax.Precision.HIGHEST))
        e_ref[:, pl.ds(s, 1)] = jnp.sum(t * b_row, axis=1, keepdims=True)

    e = e_ref[...]
    em = jnp.max(e, axis=1, keepdims=True)
    ex = jnp.exp(e - em)
    att = ex / jnp.sum(ex, axis=1, keepdims=True)

    u = att[:, 0:1] * h_ref[0]
    for s in range(1, S):
        u = u + att[:, s:s + 1] * h_ref[s]

    scores = lax.dot_general(
        u, nodes_ref[...], (((1,), (1,)), ((), ())),
        preferred_element_type=jnp.float32,
        precision=lax.Precision.HIGHEST) + bias_ref[...]
    scores_ref[...] = scores

    m = jnp.max(scores, axis=1, keepdims=True)
    lse = jnp.log(jnp.sum(jnp.exp(scores - m), axis=1, keepdims=True)) + m
    logp = scores - lse
    ids = lax.broadcasted_iota(jnp.int32, (B, N_ENTITY), 1)
    picked = jnp.where(ids == lab_ref[...], logp, 0.0)
    loss_ref[...] = jnp.full((1, 1), -jnp.sum(picked) / B, jnp.float32)


def _tail(seed, lab, agg2, root, rgcn_bias_row, attn_a, attn_b_row,
          out_bias_row):
    return pl.pallas_call(
        _tail_body,
        in_specs=[
            pl.BlockSpec(memory_space=pltpu.SMEM),
            pl.BlockSpec((B, 1), lambda: (0, 0)),
            pl.BlockSpec((2, NP, DW), lambda: (0, 0, 0)),
            pl.BlockSpec((N_ENTITY, DIM), lambda: (0, 0)),
            pl.BlockSpec((1, DIM), lambda: (0, 0)),
            pl.BlockSpec((DIM, DIM), lambda: (0, 0)),
            pl.BlockSpec((1, DIM), lambda: (0, 0)),
            pl.BlockSpec((1, N_ENTITY), lambda: (0, 0)),
        ],
        out_specs=[
            pl.BlockSpec((B, N_ENTITY), lambda: (0, 0)),
            pl.BlockSpec((1, 1), lambda: (0, 0)),
        ],
        out_shape=[
            jax.ShapeDtypeStruct((B, N_ENTITY), jnp.float32),
            jax.ShapeDtypeStruct((1, 1), jnp.float32),
        ],
        scratch_shapes=[
            pltpu.VMEM((N_ENTITY, DIM), jnp.float32),
            pltpu.VMEM((S, B, DIM), jnp.float32),
            pltpu.VMEM((B, 32), jnp.float32),
        ],
    )(seed, lab, agg2, root, rgcn_bias_row, attn_a, attn_b_row, out_bias_row)


def kernel(seed_sets, labels, edge_idx, edge_type, basis, comp, root,
           rgcn_bias, attn_a, attn_b, out_bias):
    seed = seed_sets.astype(jnp.int32)
    lab = labels.astype(jnp.int32).reshape(B, 1)
    ei = edge_idx.astype(jnp.int32).reshape(2, ER_IN, CHUNK)
    et = edge_type.astype(jnp.int32).reshape(ER_IN, CHUNK)

    w = _build_w(comp, basis).reshape(N_REL * N_ENTITY, DW)
    fidx, dst_p = _prep_edges(ei, et)

    za = jnp.zeros((NP, DW), jnp.float32)
    agg2 = _sc_agg(fidx, dst_p, w, za)

    scores, lossm = _tail(seed, lab, agg2, root, rgcn_bias.reshape(1, DIM),
                          attn_a, attn_b.reshape(1, DIM),
                          out_bias.reshape(1, N_ENTITY))
    base_loss = lossm[0, 0]
    return scores, base_loss, base_loss


# speedup vs baseline: 1.6919x; 1.6919x over previous
import functools

import jax
import jax.numpy as jnp
from jax import lax
from jax.experimental import pallas as pl
from jax.experimental.pallas import tpu as pltpu
from jax.experimental.pallas import tpu_sc as plsc

N_ENTITY = 10000
N_REL = 12
DIM = 128
NUM_BASES = 8
N_EDGES = 320000
B = 64
S = 20

NW = 32
CHUNK = 128
RPT = 80
E_PAD = NW * RPT * CHUNK
EROWS = E_PAD // CHUNK
ER_IN = N_EDGES // CHUNK
SRows = 40
SUB = 32
NBUF = 4
NP = 10112
ROWS_PER_TILE = NP // 16


def _w_body(comp_ref, basis_ref, w_ref):
    bas = basis_ref[...]
    for r in range(N_REL):
        acc = comp_ref[r, 0] * bas[0]
        for b in range(1, NUM_BASES):
            acc = acc + comp_ref[r, b] * bas[b]
        w_ref[r] = acc


def _build_w(comp, basis):
    BN = 1000
    return pl.pallas_call(
        _w_body,
        grid=(N_ENTITY // BN,),
        in_specs=[
            pl.BlockSpec(memory_space=pltpu.SMEM),
            pl.BlockSpec((NUM_BASES, BN, DIM), lambda i: (0, i, 0)),
        ],
        out_specs=pl.BlockSpec((N_REL, BN, DIM), lambda i: (0, i, 0)),
        out_shape=jax.ShapeDtypeStruct((N_REL, N_ENTITY, DIM), jnp.float32),
    )(comp, basis)


def _prep_body(ei_ref, et_ref, fidx_ref, dstp_ref):
    fidx_ref[0:ER_IN] = et_ref[...] * N_ENTITY + ei_ref[0]
    fidx_ref[ER_IN:EROWS] = jnp.zeros((EROWS - ER_IN, CHUNK), jnp.int32)
    dstp_ref[0:ER_IN] = ei_ref[1]
    dstp_ref[ER_IN:EROWS] = jnp.full((EROWS - ER_IN, CHUNK), N_ENTITY,
                                     jnp.int32)


def _prep_edges(ei, et):
    return pl.pallas_call(
        _prep_body,
        out_shape=[
            jax.ShapeDtypeStruct((EROWS, CHUNK), jnp.int32),
            jax.ShapeDtypeStruct((EROWS, CHUNK), jnp.int32),
        ],
    )(ei, et)


def _sc_agg_body(fidx_h, dst_h, w_h, za_h, zd_h, ones_h,
                 agg_out, deg_out,
                 fidx_v, dst_v,
                 fr0, fr1, fr2, fr3, dr0, dr1, dr2, dr3,
                 rw0, rw1, rw2, rw3, ones_v, agg_s, deg_s,
                 sg0, sg1, sg2, sg3, ss0, ss1, ss2, ss3,
                 sd0, sd1, sd2, sd3):
    cid = lax.axis_index("c")
    sid = lax.axis_index("s")
    wid = cid * 16 + sid

    FR = [fr0, fr1, fr2, fr3]
    DR = [dr0, dr1, dr2, dr3]
    RW = [rw0, rw1, rw2, rw3]
    SG = [sg0, sg1, sg2, sg3]
    SS = [ss0, ss1, ss2, ss3]
    SD = [sd0, sd1, sd2, sd3]

    off = sid * ROWS_PER_TILE
    pltpu.sync_copy(za_h.at[pl.ds(off, ROWS_PER_TILE)],
                    agg_s.at[pl.ds(off, ROWS_PER_TILE)])
    pltpu.sync_copy(zd_h.at[pl.ds(off, ROWS_PER_TILE)],
                    deg_s.at[pl.ds(off, ROWS_PER_TILE)])
    pltpu.sync_copy(ones_h, ones_v)

    rbase = wid * RPT
    NSUB = SRows * CHUNK // SUB

    plsc.subcore_barrier()

    def copy_idx(s2, k, cb):
        r = s2 // 4
        for j in range(SUB // 16):
            FR[k][pl.ds(16 * j, 16)] = fidx_v[r, pl.ds(cb + 16 * j, 16)]
            DR[k][pl.ds(16 * j, 16)] = dst_v[r, pl.ds(cb + 16 * j, 16)]

    def gstart(k):
        pltpu.async_copy(w_h.at[FR[k]], RW[k], SG[k])

    def gwait(k):
        pltpu.make_async_copy(w_h.at[FR[k]], RW[k], SG[k]).wait()

    def sstart(k):
        pltpu.async_copy(RW[k], agg_s.at[DR[k]], SS[k], add=True)
        pltpu.async_copy(ones_v, deg_s.at[DR[k]], SD[k], add=True)

    def swait(k):
        pltpu.make_async_copy(RW[k], agg_s.at[DR[k]], SS[k]).wait()
        pltpu.make_async_copy(ones_v, deg_s.at[DR[k]], SD[k]).wait()

    for ph in range(RPT // SRows):
        pltpu.sync_copy(fidx_h.at[pl.ds(rbase + ph * SRows, SRows)], fidx_v)
        pltpu.sync_copy(dst_h.at[pl.ds(rbase + ph * SRows, SRows)], dst_v)

        copy_idx(jnp.int32(0), 0, 0)
        gstart(0)
        copy_idx(jnp.int32(1), 1, SUB)
        gstart(1)

        def quad(q, carry):
            for k in range(NBUF):
                s = NBUF * q + k
                gwait(k)
                sstart(k)
                kp = (k + 2) % NBUF

                @pl.when(s + 2 < NSUB)
                def _():
                    @pl.when(s >= 2)
                    def _():
                        swait(kp)
                    copy_idx(s + 2, kp, kp * SUB)
                    gstart(kp)
            return carry

        lax.fori_loop(0, NSUB // NBUF, quad, 0)
        for k in range(NBUF):
            swait(k)

    plsc.subcore_barrier()

    pltpu.sync_copy(agg_s.at[pl.ds(off, ROWS_PER_TILE)],
                    agg_out.at[cid, pl.ds(off, ROWS_PER_TILE)])
    pltpu.sync_copy(deg_s.at[pl.ds(off, ROWS_PER_TILE)],
                    deg_out.at[cid, pl.ds(off, ROWS_PER_TILE)])


_sc_agg = functools.partial(
    pl.kernel,
    out_type=(
        jax.ShapeDtypeStruct((2, NP, DIM), jnp.float32),
        jax.ShapeDtypeStruct((2, NP, 16), jnp.float32),
    ),
    mesh=plsc.VectorSubcoreMesh(core_axis_name="c", subcore_axis_name="s"),
    compiler_params=pltpu.CompilerParams(use_tc_tiling_on_sc=False),
    scratch_types=(
        [
            pltpu.VMEM((SRows, CHUNK), jnp.int32),
            pltpu.VMEM((SRows, CHUNK), jnp.int32),
        ]
        + [pltpu.VMEM((SUB,), jnp.int32) for _ in range(2 * NBUF)]
        + [pltpu.VMEM((SUB, DIM), jnp.float32) for _ in range(NBUF)]
        + [
            pltpu.VMEM((SUB, 16), jnp.float32),
            pltpu.VMEM_SHARED((NP, DIM), jnp.float32),
            pltpu.VMEM_SHARED((NP, 16), jnp.float32),
        ]
        + [pltpu.SemaphoreType.DMA for _ in range(3 * NBUF)]
    ),
)(_sc_agg_body)


def _nodes_body(agg_a_ref, agg_b_ref, deg_a_ref, deg_b_ref, root_ref,
                bias_ref, out_ref):
    deg = deg_a_ref[0][:, :1] + deg_b_ref[0][:, :1]
    deg = jnp.maximum(deg, 1.0)
    out_ref[...] = ((agg_a_ref[0] + agg_b_ref[0]) / deg
                    + root_ref[...] + bias_ref[...])


def _assemble_nodes(agg2, deg2, root, rgcn_bias_row):
    BN = 1000
    return pl.pallas_call(
        _nodes_body,
        grid=(N_ENTITY // BN,),
        in_specs=[
            pl.BlockSpec((1, BN, DIM), lambda i: (0, i, 0)),
            pl.BlockSpec((1, BN, DIM), lambda i: (1, i, 0)),
            pl.BlockSpec((1, BN, 16), lambda i: (0, i, 0)),
            pl.BlockSpec((1, BN, 16), lambda i: (1, i, 0)),
            pl.BlockSpec((BN, DIM), lambda i: (i, 0)),
            pl.BlockSpec((1, DIM), lambda i: (0, 0)),
        ],
        out_specs=pl.BlockSpec((BN, DIM), lambda i: (i, 0)),
        out_shape=jax.ShapeDtypeStruct((N_ENTITY, DIM), jnp.float32),
    )(agg2, agg2, deg2, deg2, root, rgcn_bias_row)


def _tail_body(seed_ref, lab_ref, nodes_ref, a_ref, b_ref, bias_ref,
               scores_ref, loss_ref, h_ref, e_ref):
    def gather_b(b, carry):
        for s in range(S):
            idx = seed_ref[b, s]
            h_ref[s, pl.ds(b, 1), :] = nodes_ref[pl.ds(idx, 1), :]
        return carry
    lax.fori_loop(0, B, gather_b, 0)

    e_ref[...] = jnp.full((B, 32), -1e30, jnp.float32)
    a_mat = a_ref[...]
    b_row = b_ref[...]
    for s in range(S):
        hs = h_ref[s]
        t = jnp.tanh(lax.dot_general(
            hs, a_mat, (((1,), (0,)), ((), ())),
            preferred_element_type=jnp.float32,
            precision=lax.Precision.HIGHEST))
        e_ref[:, pl.ds(s, 1)] = jnp.sum(t * b_row, axis=1, keepdims=True)

    e = e_ref[...]
    em = jnp.max(e, axis=1, keepdims=True)
    ex = jnp.exp(e - em)
    att = ex / jnp.sum(ex, axis=1, keepdims=True)

    u = att[:, 0:1] * h_ref[0]
    for s in range(1, S):
        u = u + att[:, s:s + 1] * h_ref[s]

    scores = lax.dot_general(
        u, nodes_ref[...], (((1,), (1,)), ((), ())),
        preferred_element_type=jnp.float32,
        precision=lax.Precision.HIGHEST) + bias_ref[...]
    scores_ref[...] = scores

    m = jnp.max(scores, axis=1, keepdims=True)
    lse = jnp.log(jnp.sum(jnp.exp(scores - m), axis=1, keepdims=True)) + m
    logp = scores - lse
    ids = lax.broadcasted_iota(jnp.int32, (B, N_ENTITY), 1)
    picked = jnp.where(ids == lab_ref[...], logp, 0.0)
    loss_ref[...] = jnp.full((1, 1), -jnp.sum(picked) / B, jnp.float32)


def _tail(seed, lab, nodes, attn_a, attn_b_row, out_bias_row):
    return pl.pallas_call(
        _tail_body,
        in_specs=[
            pl.BlockSpec(memory_space=pltpu.SMEM),
            pl.BlockSpec((B, 1), lambda: (0, 0)),
            pl.BlockSpec((N_ENTITY, DIM), lambda: (0, 0)),
            pl.BlockSpec((DIM, DIM), lambda: (0, 0)),
            pl.BlockSpec((1, DIM), lambda: (0, 0)),
            pl.BlockSpec((1, N_ENTITY), lambda: (0, 0)),
        ],
        out_specs=[
            pl.BlockSpec((B, N_ENTITY), lambda: (0, 0)),
            pl.BlockSpec((1, 1), lambda: (0, 0)),
        ],
        out_shape=[
            jax.ShapeDtypeStruct((B, N_ENTITY), jnp.float32),
            jax.ShapeDtypeStruct((1, 1), jnp.float32),
        ],
        scratch_shapes=[
            pltpu.VMEM((S, B, DIM), jnp.float32),
            pltpu.VMEM((B, 32), jnp.float32),
        ],
    )(seed, lab, nodes, attn_a, attn_b_row, out_bias_row)


def kernel(seed_sets, labels, edge_idx, edge_type, basis, comp, root,
           rgcn_bias, attn_a, attn_b, out_bias):
    seed = seed_sets.astype(jnp.int32)
    lab = labels.astype(jnp.int32).reshape(B, 1)
    ei = edge_idx.astype(jnp.int32).reshape(2, ER_IN, CHUNK)
    et = edge_type.astype(jnp.int32).reshape(ER_IN, CHUNK)

    w = _build_w(comp, basis).reshape(N_REL * N_ENTITY, DIM)
    fidx, dst_p = _prep_edges(ei, et)

    za = jnp.zeros((NP, DIM), jnp.float32)
    zd = jnp.zeros((NP, 16), jnp.float32)
    ones_b = jnp.ones((SUB, 16), jnp.float32)
    agg2, deg2 = _sc_agg(fidx, dst_p, w, za, zd, ones_b)

    nodes = _assemble_nodes(agg2, deg2, root, rgcn_bias.reshape(1, DIM))

    scores, lossm = _tail(seed, lab, nodes, attn_a,
                          attn_b.reshape(1, DIM), out_bias.reshape(1, N_ENTITY))
    base_loss = lossm[0, 0]
    return scores, base_loss, base_loss


# speedup vs baseline: 1.7000x; 1.0048x over previous
import functools

import jax
import jax.numpy as jnp
from jax import lax
from jax.experimental import pallas as pl
from jax.experimental.pallas import tpu as pltpu
from jax.experimental.pallas import tpu_sc as plsc

N_ENTITY = 10000
N_REL = 12
DIM = 128
NUM_BASES = 8
N_EDGES = 320000
B = 64
S = 20

NW = 32
CHUNK = 128
RPT = 80
E_PAD = NW * RPT * CHUNK
EROWS = E_PAD // CHUNK
ER_IN = N_EDGES // CHUNK
SRows = 16
SUB = 64
SPR = CHUNK // SUB
NBUF = 4
NP = 10112
ROWS_PER_TILE = NP // 16


def _w_body(comp_ref, basis_ref, w_ref):
    bas = basis_ref[...]
    for r in range(N_REL):
        acc = comp_ref[r, 0] * bas[0]
        for b in range(1, NUM_BASES):
            acc = acc + comp_ref[r, b] * bas[b]
        w_ref[r] = acc


def _build_w(comp, basis):
    BN = 1000
    return pl.pallas_call(
        _w_body,
        grid=(N_ENTITY // BN,),
        in_specs=[
            pl.BlockSpec(memory_space=pltpu.SMEM),
            pl.BlockSpec((NUM_BASES, BN, DIM), lambda i: (0, i, 0)),
        ],
        out_specs=pl.BlockSpec((N_REL, BN, DIM), lambda i: (0, i, 0)),
        out_shape=jax.ShapeDtypeStruct((N_REL, N_ENTITY, DIM), jnp.float32),
    )(comp, basis)


def _prep_body(ei_ref, et_ref, fidx_ref, dstp_ref):
    fidx_ref[0:ER_IN] = et_ref[...] * N_ENTITY + ei_ref[0]
    fidx_ref[ER_IN:EROWS] = jnp.zeros((EROWS - ER_IN, CHUNK), jnp.int32)
    dstp_ref[0:ER_IN] = ei_ref[1]
    dstp_ref[ER_IN:EROWS] = jnp.full((EROWS - ER_IN, CHUNK), N_ENTITY,
                                     jnp.int32)


def _prep_edges(ei, et):
    return pl.pallas_call(
        _prep_body,
        out_shape=[
            jax.ShapeDtypeStruct((EROWS, CHUNK), jnp.int32),
            jax.ShapeDtypeStruct((EROWS, CHUNK), jnp.int32),
        ],
    )(ei, et)


def _sc_agg_body(fidx_h, dst_h, w_h, za_h, zd_h, ones_h,
                 agg_out, deg_out,
                 fidx_v, dst_v,
                 fr0, fr1, fr2, fr3, dr0, dr1, dr2, dr3,
                 rw0, rw1, rw2, rw3, ones_v, agg_s, deg_s,
                 sg0, sg1, sg2, sg3, ss0, ss1, ss2, ss3,
                 sd0, sd1, sd2, sd3):
    cid = lax.axis_index("c")
    sid = lax.axis_index("s")
    wid = cid * 16 + sid

    FR = [fr0, fr1, fr2, fr3]
    DR = [dr0, dr1, dr2, dr3]
    RW = [rw0, rw1, rw2, rw3]
    SG = [sg0, sg1, sg2, sg3]
    SS = [ss0, ss1, ss2, ss3]
    SD = [sd0, sd1, sd2, sd3]

    off = sid * ROWS_PER_TILE
    pltpu.sync_copy(za_h.at[pl.ds(off, ROWS_PER_TILE)],
                    agg_s.at[pl.ds(off, ROWS_PER_TILE)])
    pltpu.sync_copy(zd_h.at[pl.ds(off, ROWS_PER_TILE)],
                    deg_s.at[pl.ds(off, ROWS_PER_TILE)])
    pltpu.sync_copy(ones_h, ones_v)

    rbase = wid * RPT
    NSUB = SRows * CHUNK // SUB

    plsc.subcore_barrier()

    def copy_idx(s2, k, cb):
        r = s2 // SPR
        for j in range(SUB // 16):
            FR[k][pl.ds(16 * j, 16)] = fidx_v[r, pl.ds(cb + 16 * j, 16)]
            DR[k][pl.ds(16 * j, 16)] = dst_v[r, pl.ds(cb + 16 * j, 16)]

    def gstart(k):
        pltpu.async_copy(w_h.at[FR[k]], RW[k], SG[k])

    def gwait(k):
        pltpu.make_async_copy(w_h.at[FR[k]], RW[k], SG[k]).wait()

    def sstart(k):
        pltpu.async_copy(RW[k], agg_s.at[DR[k]], SS[k], add=True)
        pltpu.async_copy(ones_v, deg_s.at[DR[k]], SD[k], add=True)

    def swait(k):
        pltpu.make_async_copy(RW[k], agg_s.at[DR[k]], SS[k]).wait()
        pltpu.make_async_copy(ones_v, deg_s.at[DR[k]], SD[k]).wait()

    for ph in range(RPT // SRows):
        pltpu.sync_copy(fidx_h.at[pl.ds(rbase + ph * SRows, SRows)], fidx_v)
        pltpu.sync_copy(dst_h.at[pl.ds(rbase + ph * SRows, SRows)], dst_v)

        copy_idx(jnp.int32(0), 0, 0)
        gstart(0)
        copy_idx(jnp.int32(1), 1, SUB)
        gstart(1)

        def quad(q, carry):
            for k in range(NBUF):
                s = NBUF * q + k
                gwait(k)
                sstart(k)
                kp = (k + 2) % NBUF

                @pl.when(s + 2 < NSUB)
                def _():
                    @pl.when(s >= 2)
                    def _():
                        swait(kp)
                    copy_idx(s + 2, kp, (kp % SPR) * SUB)
                    gstart(kp)
            return carry

        lax.fori_loop(0, NSUB // NBUF, quad, 0)
        for k in range(NBUF):
            swait(k)

    plsc.subcore_barrier()

    pltpu.sync_copy(agg_s.at[pl.ds(off, ROWS_PER_TILE)],
                    agg_out.at[cid, pl.ds(off, ROWS_PER_TILE)])
    pltpu.sync_copy(deg_s.at[pl.ds(off, ROWS_PER_TILE)],
                    deg_out.at[cid, pl.ds(off, ROWS_PER_TILE)])


_sc_agg = functools.partial(
    pl.kernel,
    out_type=(
        jax.ShapeDtypeStruct((2, NP, DIM), jnp.float32),
        jax.ShapeDtypeStruct((2, NP, 16), jnp.float32),
    ),
    mesh=plsc.VectorSubcoreMesh(core_axis_name="c", subcore_axis_name="s"),
    compiler_params=pltpu.CompilerParams(use_tc_tiling_on_sc=False),
    scratch_types=(
        [
            pltpu.VMEM((SRows, CHUNK), jnp.int32),
            pltpu.VMEM((SRows, CHUNK), jnp.int32),
        ]
        + [pltpu.VMEM((SUB,), jnp.int32) for _ in range(2 * NBUF)]
        + [pltpu.VMEM((SUB, DIM), jnp.float32) for _ in range(NBUF)]
        + [
            pltpu.VMEM((SUB, 16), jnp.float32),
            pltpu.VMEM_SHARED((NP, DIM), jnp.float32),
            pltpu.VMEM_SHARED((NP, 16), jnp.float32),
        ]
        + [pltpu.SemaphoreType.DMA for _ in range(3 * NBUF)]
    ),
)(_sc_agg_body)


def _nodes_body(agg_a_ref, agg_b_ref, deg_a_ref, deg_b_ref, root_ref,
                bias_ref, out_ref):
    deg = deg_a_ref[0][:, :1] + deg_b_ref[0][:, :1]
    deg = jnp.maximum(deg, 1.0)
    out_ref[...] = ((agg_a_ref[0] + agg_b_ref[0]) / deg
                    + root_ref[...] + bias_ref[...])


def _assemble_nodes(agg2, deg2, root, rgcn_bias_row):
    BN = 1000
    return pl.pallas_call(
        _nodes_body,
        grid=(N_ENTITY // BN,),
        in_specs=[
            pl.BlockSpec((1, BN, DIM), lambda i: (0, i, 0)),
            pl.BlockSpec((1, BN, DIM), lambda i: (1, i, 0)),
            pl.BlockSpec((1, BN, 16), lambda i: (0, i, 0)),
            pl.BlockSpec((1, BN, 16), lambda i: (1, i, 0)),
            pl.BlockSpec((BN, DIM), lambda i: (i, 0)),
            pl.BlockSpec((1, DIM), lambda i: (0, 0)),
        ],
        out_specs=pl.BlockSpec((BN, DIM), lambda i: (i, 0)),
        out_shape=jax.ShapeDtypeStruct((N_ENTITY, DIM), jnp.float32),
    )(agg2, agg2, deg2, deg2, root, rgcn_bias_row)


def _tail_body(seed_ref, lab_ref, nodes_ref, a_ref, b_ref, bias_ref,
               scores_ref, loss_ref, h_ref, e_ref):
    def gather_b(b, carry):
        for s in range(S):
            idx = seed_ref[b, s]
            h_ref[s, pl.ds(b, 1), :] = nodes_ref[pl.ds(idx, 1), :]
        return carry
    lax.fori_loop(0, B, gather_b, 0)

    e_ref[...] = jnp.full((B, 32), -1e30, jnp.float32)
    a_mat = a_ref[...]
    b_row = b_ref[...]
    for s in range(S):
        hs = h_ref[s]
        t = jnp.tanh(lax.dot_general(
            hs, a_mat, (((1,), (0,)), ((), ())),
            preferred_element_type=jnp.float32,
            precision=lax.Precision.HIGHEST))
        e_ref[:, pl.ds(s, 1)] = jnp.sum(t * b_row, axis=1, keepdims=True)

    e = e_ref[...]
    em = jnp.max(e, axis=1, keepdims=True)
    ex = jnp.exp(e - em)
    att = ex / jnp.sum(ex, axis=1, keepdims=True)

    u = att[:, 0:1] * h_ref[0]
    for s in range(1, S):
        u = u + att[:, s:s + 1] * h_ref[s]

    scores = lax.dot_general(
        u, nodes_ref[...], (((1,), (1,)), ((), ())),
        preferred_element_type=jnp.float32,
        precision=lax.Precision.HIGHEST) + bias_ref[...]
    scores_ref[...] = scores

    m = jnp.max(scores, axis=1, keepdims=True)
    lse = jnp.log(jnp.sum(jnp.exp(scores - m), axis=1, keepdims=True)) + m
    logp = scores - lse
    ids = lax.broadcasted_iota(jnp.int32, (B, N_ENTITY), 1)
    picked = jnp.where(ids == lab_ref[...], logp, 0.0)
    loss_ref[...] = jnp.full((1, 1), -jnp.sum(picked) / B, jnp.float32)


def _tail(seed, lab, nodes, attn_a, attn_b_row, out_bias_row):
    return pl.pallas_call(
        _tail_body,
        in_specs=[
            pl.BlockSpec(memory_space=pltpu.SMEM),
            pl.BlockSpec((B, 1), lambda: (0, 0)),
            pl.BlockSpec((N_ENTITY, DIM), lambda: (0, 0)),
            pl.BlockSpec((DIM, DIM), lambda: (0, 0)),
            pl.BlockSpec((1, DIM), lambda: (0, 0)),
            pl.BlockSpec((1, N_ENTITY), lambda: (0, 0)),
        ],
        out_specs=[
            pl.BlockSpec((B, N_ENTITY), lambda: (0, 0)),
            pl.BlockSpec((1, 1), lambda: (0, 0)),
        ],
        out_shape=[
            jax.ShapeDtypeStruct((B, N_ENTITY), jnp.float32),
            jax.ShapeDtypeStruct((1, 1), jnp.float32),
        ],
        scratch_shapes=[
            pltpu.VMEM((S, B, DIM), jnp.float32),
            pltpu.VMEM((B, 32), jnp.float32),
        ],
    )(seed, lab, nodes, attn_a, attn_b_row, out_bias_row)


def kernel(seed_sets, labels, edge_idx, edge_type, basis, comp, root,
           rgcn_bias, attn_a, attn_b, out_bias):
    seed = seed_sets.astype(jnp.int32)
    lab = labels.astype(jnp.int32).reshape(B, 1)
    ei = edge_idx.astype(jnp.int32).reshape(2, ER_IN, CHUNK)
    et = edge_type.astype(jnp.int32).reshape(ER_IN, CHUNK)

    w = _build_w(comp, basis).reshape(N_REL * N_ENTITY, DIM)
    fidx, dst_p = _prep_edges(ei, et)

    za = jnp.zeros((NP, DIM), jnp.float32)
    zd = jnp.zeros((NP, 16), jnp.float32)
    ones_b = jnp.ones((SUB, 16), jnp.float32)
    agg2, deg2 = _sc_agg(fidx, dst_p, w, za, zd, ones_b)

    nodes = _assemble_nodes(agg2, deg2, root, rgcn_bias.reshape(1, DIM))

    scores, lossm = _tail(seed, lab, nodes, attn_a,
                          attn_b.reshape(1, DIM), out_bias.reshape(1, N_ENTITY))
    base_loss = lossm[0, 0]
    return scores, base_loss, base_loss
